# baseline (speedup 1.0000x reference)
import functools

import jax
import jax.numpy as jnp
from jax import lax
from jax.experimental import pallas as pl
from jax.experimental.pallas import tpu as pltpu
from jax.experimental.pallas import tpu_sc as plsc

VOCAB = 1000000
DIM = 64
BATCH = 16384
HIST = 200

N = BATCH * HIST
NC, NS = 2, 16
NW = NC * NS
PER_W = N // NW
IDXW = 128
G = 8
CHUNK = G * IDXW
NCHUNK = PER_W // CHUNK

_mesh = plsc.VectorSubcoreMesh(core_axis_name="c", subcore_axis_name="s")


@functools.partial(
    pl.kernel,
    mesh=_mesh,
    out_type=jax.ShapeDtypeStruct((N, DIM), jnp.float32),
    scratch_types=[
        pltpu.VMEM((G, IDXW), jnp.int32),
        pltpu.VMEM((CHUNK, DIM), jnp.float32),
        pltpu.SemaphoreType.DMA,
    ],
    compiler_params=pltpu.CompilerParams(use_tc_tiling_on_sc=False),
)
def _emb_gather(x_hbm, table_hbm, out_hbm, idx_v, rows_v, sem):
    wid = lax.axis_index("s") * NC + lax.axis_index("c")
    base = wid * PER_W

    def body(g, carry):
        row0 = pl.multiple_of(base + g * CHUNK, CHUNK)
        pltpu.sync_copy(x_hbm.at[pl.ds(pl.multiple_of(row0 // IDXW, G), G)], idx_v)
        cps = [
            pltpu.async_copy(
                table_hbm.at[idx_v.at[j]],
                rows_v.at[pl.ds(j * IDXW, IDXW)],
                sem,
            )
            for j in range(G)
        ]
        for cp in cps:
            cp.wait()
        pltpu.sync_copy(rows_v, out_hbm.at[pl.ds(row0, CHUNK)])
        return carry

    lax.fori_loop(0, NCHUNK, body, 0)


def kernel(x, table):
    x2d = x.astype(jnp.int32).reshape(N // IDXW, IDXW)
    out = _emb_gather(x2d, table)
    return out.reshape(BATCH, HIST, DIM)


# speedup vs baseline: 1.0309x; 1.0309x over previous
import functools

import jax
import jax.numpy as jnp
from jax import lax
from jax.experimental import pallas as pl
from jax.experimental.pallas import tpu as pltpu
from jax.experimental.pallas import tpu_sc as plsc

VOCAB = 1000000
DIM = 64
BATCH = 16384
HIST = 200

N = BATCH * HIST
NC, NS = 2, 16
NW = NC * NS
PER_W = N // NW
IDXW = 128
G = 4
HALF = G * IDXW
SUP = 2 * HALF
NSUP = PER_W // SUP

_mesh = plsc.VectorSubcoreMesh(core_axis_name="c", subcore_axis_name="s")


@functools.partial(
    pl.kernel,
    mesh=_mesh,
    out_type=jax.ShapeDtypeStruct((N, DIM), jnp.float32),
    scratch_types=[
        pltpu.VMEM((2, 8, IDXW), jnp.int32),
        pltpu.VMEM((2, HALF, DIM), jnp.float32),
        pltpu.SemaphoreType.DMA,
        pltpu.SemaphoreType.DMA,
        pltpu.SemaphoreType.DMA,
    ],
    compiler_params=pltpu.CompilerParams(use_tc_tiling_on_sc=False),
)
def _emb_gather(x_hbm, table_hbm, out_hbm, idx_v, rows_v, isem, gsem, wsem):
    wid = lax.axis_index("s") * NC + lax.axis_index("c")
    base = wid * PER_W
    xrow0 = base // IDXW

    def idx_src(s):
        return x_hbm.at[pl.ds(pl.multiple_of(xrow0 + s * 8, 8), 8)]

    def fire_idx(s, slot):
        pltpu.async_copy(idx_src(s), idx_v.at[slot], isem)

    def wait_idx(slot):
        pltpu.make_async_copy(idx_src(0), idx_v.at[slot], isem).wait()

    def fire_g(slot, hh, b):
        for j in range(G):
            pltpu.async_copy(
                table_hbm.at[idx_v.at[slot, G * hh + j]],
                rows_v.at[b, pl.ds(j * IDXW, IDXW)],
                gsem,
            )

    def wait_g(b):
        for j in range(G):
            pltpu.make_async_copy(
                table_hbm.at[idx_v.at[0, j]],
                rows_v.at[b, pl.ds(j * IDXW, IDXW)],
                gsem,
            ).wait()

    def fire_wb(h, b):
        row0 = pl.multiple_of(base + h * HALF, HALF)
        pltpu.async_copy(rows_v.at[b], out_hbm.at[pl.ds(row0, HALF)], wsem)

    def wait_wb(b):
        pltpu.make_async_copy(
            rows_v.at[b], out_hbm.at[pl.ds(0, HALF)], wsem
        ).wait()

    pltpu.sync_copy(idx_src(0), idx_v.at[0])
    fire_idx(1, 1)
    fire_g(0, 0, 0)

    def body(i, carry):
        s0 = 2 * i
        h0 = 4 * i

        @pl.when(i > 0)
        def _():
            wait_wb(1)
        fire_g(0, 1, 1)
        wait_g(0)
        fire_wb(h0, 0)

        wait_idx(1)
        wait_wb(0)
        fire_g(1, 0, 0)
        wait_g(1)
        fire_idx(jnp.minimum(s0 + 2, NSUP - 1), 0)
        fire_wb(h0 + 1, 1)

        wait_wb(1)
        fire_g(1, 1, 1)
        wait_g(0)
        fire_wb(h0 + 2, 0)

        wait_wb(0)
        wait_idx(0)
        fire_g(0, 0, 0)
        wait_g(1)
        fire_wb(h0 + 3, 1)
        fire_idx(jnp.minimum(s0 + 3, NSUP - 1), 1)
        return carry

    lax.fori_loop(0, NSUP // 2, body, 0)

    wait_wb(1)
    wait_g(0)
    wait_idx(1)


def kernel(x, table):
    x2d = x.astype(jnp.int32).reshape(N // IDXW, IDXW)
    out = _emb_gather(x2d, table)
    return out.reshape(BATCH, HIST, DIM)
